# baseline (speedup 1.0000x reference)
import jax
import jax.numpy as jnp
from jax import lax
from jax.experimental import pallas as pl
from jax.experimental.pallas import tpu as pltpu
from jax.experimental.pallas import tpu_sc as plsc

_STM = 2048
_EMB = 2048
_NW = 32
_ROWS = _STM // _NW


def _stripe_copy(layer_hbm, mem_hbm, out_hbm, layer_s):
    c = lax.axis_index("c")
    s = lax.axis_index("s")
    wid = s * 2 + c
    pltpu.sync_copy(layer_hbm, layer_s)
    layer = layer_s[...][0]
    base = wid * _ROWS
    pltpu.sync_copy(mem_hbm.at[layer, pl.ds(base, _ROWS)],
                    out_hbm.at[0, pl.ds(base, _ROWS)])


_sc_copy = pl.kernel(
    _stripe_copy,
    out_type=jax.ShapeDtypeStruct((1, _STM, _EMB), jnp.float32),
    mesh=plsc.VectorSubcoreMesh(core_axis_name="c", subcore_axis_name="s"),
    scratch_types=[pltpu.VMEM((16,), jnp.int32)],
)


def kernel(memory, layer):
    return _sc_copy(jnp.full((16,), layer, dtype=jnp.int32), memory)
